# baseline (device time: 194335 ns/iter reference)
import jax
import jax.numpy as jnp
import numpy as np
from jax import lax
from jax.experimental import pallas as pl
from jax.experimental.pallas import tpu as pltpu

N_DEV = 4
SQ = 1024
D = 1024
HQ = 8
DH = 128
SQH = SQ // 2
SCALE = 0.08838834764831843


def _rope_tables():
    inv = 1.0 / (10000.0 ** (np.arange(0, DH, 2) / DH))
    pos = np.arange(SQ)[:, None] * inv[None, :]
    cos = np.repeat(np.cos(pos), 2, axis=-1).astype(np.float32)
    sin = np.repeat(np.sin(pos), 2, axis=-1).astype(np.float32)
    cos_t = np.tile(cos, (1, HQ))
    sin_t = np.tile(sin, (1, HQ))
    P = np.zeros((DH, DH), np.float32)
    for k in range(DH // 2):
        P[2 * k + 1, 2 * k] = -1.0
        P[2 * k, 2 * k + 1] = 1.0
    P_full = np.kron(np.eye(HQ, dtype=np.float32), P)
    return cos_t, sin_t, P_full


def _body(x_ref, wq_ref, wk_ref, wv_ref, wo_ref, cos_ref, sin_ref, p_ref,
          out_ref, xbuf, comm, ag_send, ag_recv, rs_send, rs_recv):
    i = lax.axis_index("i")
    right = lax.rem(i + 1, N_DEV)
    left = lax.rem(i + N_DEV - 1, N_DEV)

    barrier = pltpu.get_barrier_semaphore()
    for nbr in (left, right):
        pl.semaphore_signal(barrier, inc=1, device_id=(nbr,),
                            device_id_type=pl.DeviceIdType.MESH)
    pl.semaphore_wait(barrier, 2)

    ag = [
        pltpu.make_async_remote_copy(
            src_ref=xbuf.at[h],
            dst_ref=xbuf.at[h + 1],
            send_sem=ag_send.at[h],
            recv_sem=ag_recv.at[h],
            device_id=(right,),
            device_id_type=pl.DeviceIdType.MESH,
        )
        for h in range(N_DEV - 1)
    ]
    rs = [
        pltpu.make_async_remote_copy(
            src_ref=xbuf.at[h + 1],
            dst_ref=comm.at[h],
            send_sem=rs_send.at[h],
            recv_sem=rs_recv.at[h],
            device_id=(right,),
            device_id_type=pl.DeviceIdType.MESH,
        )
        for h in range(N_DEV - 2)
    ]
    rs_last = [
        pltpu.make_async_remote_copy(
            src_ref=xbuf.at[3, pl.ds(half * SQH, SQH)],
            dst_ref=comm.at[2, pl.ds(half * SQH, SQH)],
            send_sem=rs_send.at[2 + half],
            recv_sem=rs_recv.at[2 + half],
            device_id=(right,),
            device_id_type=pl.DeviceIdType.MESH,
        )
        for half in range(2)
    ]

    cos = cos_ref[...]
    sin = sin_ref[...]
    wqp = jnp.dot(wq_ref[...], p_ref[...],
                  preferred_element_type=jnp.float32).astype(jnp.bfloat16)
    wkp = jnp.dot(wk_ref[...], p_ref[...],
                  preferred_element_type=jnp.float32).astype(jnp.bfloat16)

    def qkv(xb):
        q = jnp.dot(xb, wq_ref[...],
                    preferred_element_type=jnp.float32).astype(jnp.bfloat16)
        qr = jnp.dot(xb, wqp,
                     preferred_element_type=jnp.float32).astype(jnp.bfloat16)
        k = jnp.dot(xb, wk_ref[...],
                    preferred_element_type=jnp.float32).astype(jnp.bfloat16)
        kr = jnp.dot(xb, wkp,
                     preferred_element_type=jnp.float32).astype(jnp.bfloat16)
        v = jnp.dot(xb, wv_ref[...],
                    preferred_element_type=jnp.float32).astype(jnp.bfloat16)
        qrope = q * cos + qr * sin
        krope = k * cos + kr * sin
        return qrope, krope, v

    def attend(qrope, krope, v, rows):
        ctx_cols = []
        for h in range(HQ):
            sl = slice(h * DH, (h + 1) * DH)
            s_mat = lax.dot_general(
                qrope[rows, sl], krope[:, sl], (((1,), (1,)), ((), ())),
                preferred_element_type=jnp.float32) * SCALE
            w = jnp.exp(s_mat)
            w = (w / jnp.sum(w, axis=1, keepdims=True)).astype(jnp.bfloat16)
            ctx_cols.append(
                jnp.dot(w, v[:, sl], preferred_element_type=jnp.float32)
                .astype(jnp.bfloat16))
        ctx = jnp.concatenate(ctx_cols, axis=1)
        return jnp.dot(ctx, wo_ref[...], preferred_element_type=jnp.float32)

    def compute_slot(xb):
        qrope, krope, v = qkv(xb)
        return attend(qrope, krope, v, slice(None))

    xbuf[0] = x_ref[0]
    ag[0].start()
    p0 = compute_slot(xbuf[0])
    ag[0].wait_send()
    xbuf[0] = p0.astype(jnp.bfloat16)

    ag[0].wait_recv()
    ag[1].start()
    p1 = compute_slot(xbuf[1])
    ag[1].wait_send()
    xbuf[1] = p1.astype(jnp.bfloat16)
    rs[0].start()

    ag[1].wait_recv()
    ag[2].start()
    p2 = compute_slot(xbuf[2])
    rs[0].wait_recv()
    acc2 = p2 + comm[0].astype(jnp.float32)
    ag[2].wait_send()
    xbuf[2] = acc2.astype(jnp.bfloat16)
    rs[1].start()

    ag[2].wait_recv()
    qrope3, krope3, v3 = qkv(xbuf[3])
    rs[1].wait_recv()
    c1 = comm[1].astype(jnp.float32)
    p3a = attend(qrope3, krope3, v3, slice(0, SQH))
    xbuf[3, pl.ds(0, SQH)] = (p3a + c1[:SQH]).astype(jnp.bfloat16)
    rs_last[0].start()
    p3b = attend(qrope3, krope3, v3, slice(SQH, SQ))
    xbuf[3, pl.ds(SQH, SQH)] = (p3b + c1[SQH:]).astype(jnp.bfloat16)
    rs_last[1].start()

    rs_last[0].wait_recv()
    rs_last[1].wait_recv()
    out_ref[0, :, :] = (xbuf[0].astype(jnp.float32)
                        + comm[2].astype(jnp.float32))

    rs[0].wait_send()
    rs[1].wait_send()
    rs_last[0].wait_send()
    rs_last[1].wait_send()


def kernel(x, Wq, Wk, Wv, Wo):
    cos_t, sin_t, p_full = _rope_tables()
    return pl.pallas_call(
        _body,
        out_shape=jax.ShapeDtypeStruct((1, SQ, D), jnp.float32),
        in_specs=[pl.BlockSpec(memory_space=pltpu.VMEM)] * 8,
        out_specs=pl.BlockSpec(memory_space=pltpu.VMEM),
        scratch_shapes=[
            pltpu.VMEM((N_DEV, SQ, D), jnp.bfloat16),
            pltpu.VMEM((N_DEV - 1, SQ, D), jnp.bfloat16),
            pltpu.SemaphoreType.DMA((N_DEV - 1,)),
            pltpu.SemaphoreType.DMA((N_DEV - 1,)),
            pltpu.SemaphoreType.DMA((N_DEV,)),
            pltpu.SemaphoreType.DMA((N_DEV,)),
        ],
        compiler_params=pltpu.CompilerParams(
            collective_id=0,
            vmem_limit_bytes=100 * 1024 * 1024,
        ),
    )(
        x.astype(jnp.bfloat16),
        Wq.astype(jnp.bfloat16),
        Wk.astype(jnp.bfloat16),
        Wv.astype(jnp.bfloat16),
        Wo.astype(jnp.bfloat16),
        jnp.asarray(cos_t, dtype=jnp.bfloat16),
        jnp.asarray(sin_t, dtype=jnp.bfloat16),
        jnp.asarray(p_full, dtype=jnp.bfloat16),
    )


# device time: 181684 ns/iter; 1.0696x vs baseline; 1.0696x over previous
import jax
import jax.numpy as jnp
import numpy as np
from jax import lax
from jax.experimental import pallas as pl
from jax.experimental.pallas import tpu as pltpu

N_DEV = 4
SQ = 1024
D = 1024
HQ = 8
DH = 128
SQH = SQ // 2
SCALE = 0.08838834764831843


def _rope_tables():
    inv = 1.0 / (10000.0 ** (np.arange(0, DH, 2) / DH))
    pos = np.arange(SQ)[:, None] * inv[None, :]
    cos = np.repeat(np.cos(pos), 2, axis=-1).astype(np.float32)
    sin = np.repeat(np.sin(pos), 2, axis=-1).astype(np.float32)
    P = np.zeros((DH, DH), np.float32)
    for k in range(DH // 2):
        P[2 * k + 1, 2 * k] = -1.0
        P[2 * k, 2 * k + 1] = 1.0
    return cos, sin, P


def _body(x_ref, wq_ref, wk_ref, wv_ref, wo_ref, cos_ref, sin_ref, p_ref,
          out_ref, xbuf, comm, ag_send, ag_recv, rs_send, rs_recv):
    i = lax.axis_index("i")
    right = lax.rem(i + 1, N_DEV)
    left = lax.rem(i + N_DEV - 1, N_DEV)

    barrier = pltpu.get_barrier_semaphore()
    for nbr in (left, right):
        pl.semaphore_signal(barrier, inc=1, device_id=(nbr,),
                            device_id_type=pl.DeviceIdType.MESH)
    pl.semaphore_wait(barrier, 2)

    ag = [
        pltpu.make_async_remote_copy(
            src_ref=xbuf.at[h],
            dst_ref=xbuf.at[h + 1],
            send_sem=ag_send.at[h],
            recv_sem=ag_recv.at[h],
            device_id=(right,),
            device_id_type=pl.DeviceIdType.MESH,
        )
        for h in range(N_DEV - 1)
    ]
    rs = [
        pltpu.make_async_remote_copy(
            src_ref=xbuf.at[h + 1],
            dst_ref=comm.at[h],
            send_sem=rs_send.at[h],
            recv_sem=rs_recv.at[h],
            device_id=(right,),
            device_id_type=pl.DeviceIdType.MESH,
        )
        for h in range(N_DEV - 2)
    ]
    rs_last = [
        pltpu.make_async_remote_copy(
            src_ref=xbuf.at[3, pl.ds(half * SQH, SQH)],
            dst_ref=comm.at[2, pl.ds(half * SQH, SQH)],
            send_sem=rs_send.at[2 + half],
            recv_sem=rs_recv.at[2 + half],
            device_id=(right,),
            device_id_type=pl.DeviceIdType.MESH,
        )
        for half in range(2)
    ]

    cos = cos_ref[...]
    sin = sin_ref[...]
    p = p_ref[...].astype(jnp.bfloat16)

    def rope_qkv(xb):
        q = jnp.dot(xb, wq_ref[...], preferred_element_type=jnp.float32)
        k = jnp.dot(xb, wk_ref[...], preferred_element_type=jnp.float32)
        v = jnp.dot(xb, wv_ref[...], preferred_element_type=jnp.float32)
        v = v.astype(jnp.bfloat16)
        qhs, khs = [], []
        for h in range(HQ):
            sl = slice(h * DH, (h + 1) * DH)
            qh32, kh32 = q[:, sl], k[:, sl]
            q_rot = jnp.dot(qh32.astype(jnp.bfloat16), p,
                            preferred_element_type=jnp.float32)
            k_rot = jnp.dot(kh32.astype(jnp.bfloat16), p,
                            preferred_element_type=jnp.float32)
            qhs.append((qh32 * cos + q_rot * sin).astype(jnp.bfloat16))
            khs.append((kh32 * cos + k_rot * sin).astype(jnp.bfloat16))
        return qhs, khs, v

    def attend(qhs, khs, v, rows, nrows):
        partial = jnp.zeros((nrows, D), jnp.float32)
        for h in range(HQ):
            sl = slice(h * DH, (h + 1) * DH)
            s_mat = lax.dot_general(
                qhs[h][rows], khs[h], (((1,), (1,)), ((), ())),
                preferred_element_type=jnp.float32) * SCALE
            w = jnp.exp(s_mat)
            w = (w / jnp.sum(w, axis=1, keepdims=True)).astype(jnp.bfloat16)
            ctx = jnp.dot(w, v[:, sl], preferred_element_type=jnp.float32)
            partial = partial + jnp.dot(ctx.astype(jnp.bfloat16),
                                        wo_ref[sl, :],
                                        preferred_element_type=jnp.float32)
        return partial

    def compute_slot(xb):
        qhs, khs, v = rope_qkv(xb)
        return attend(qhs, khs, v, slice(None), SQ)

    xbuf[0] = x_ref[0]
    ag[0].start()
    p0 = compute_slot(xbuf[0])
    ag[0].wait_send()
    xbuf[0] = p0.astype(jnp.bfloat16)

    ag[0].wait_recv()
    ag[1].start()
    p1 = compute_slot(xbuf[1])
    ag[1].wait_send()
    xbuf[1] = p1.astype(jnp.bfloat16)
    rs[0].start()

    ag[1].wait_recv()
    ag[2].start()
    p2 = compute_slot(xbuf[2])
    rs[0].wait_recv()
    acc2 = p2 + comm[0].astype(jnp.float32)
    ag[2].wait_send()
    xbuf[2] = acc2.astype(jnp.bfloat16)
    rs[1].start()

    ag[2].wait_recv()
    qhs3, khs3, v3 = rope_qkv(xbuf[3])
    rs[1].wait_recv()
    c1 = comm[1].astype(jnp.float32)
    p3a = attend(qhs3, khs3, v3, slice(0, SQH), SQH)
    xbuf[3, pl.ds(0, SQH)] = (p3a + c1[:SQH]).astype(jnp.bfloat16)
    rs_last[0].start()
    p3b = attend(qhs3, khs3, v3, slice(SQH, SQ), SQH)
    xbuf[3, pl.ds(SQH, SQH)] = (p3b + c1[SQH:]).astype(jnp.bfloat16)
    rs_last[1].start()

    rs_last[0].wait_recv()
    rs_last[1].wait_recv()
    out_ref[0, :, :] = (xbuf[0].astype(jnp.float32)
                        + comm[2].astype(jnp.float32))

    rs[0].wait_send()
    rs[1].wait_send()
    rs_last[0].wait_send()
    rs_last[1].wait_send()


def kernel(x, Wq, Wk, Wv, Wo):
    cos_np, sin_np, p_np = _rope_tables()
    return pl.pallas_call(
        _body,
        out_shape=jax.ShapeDtypeStruct((1, SQ, D), jnp.float32),
        in_specs=[pl.BlockSpec(memory_space=pltpu.VMEM)] * 8,
        out_specs=pl.BlockSpec(memory_space=pltpu.VMEM),
        scratch_shapes=[
            pltpu.VMEM((N_DEV, SQ, D), jnp.bfloat16),
            pltpu.VMEM((N_DEV - 1, SQ, D), jnp.bfloat16),
            pltpu.SemaphoreType.DMA((N_DEV - 1,)),
            pltpu.SemaphoreType.DMA((N_DEV - 1,)),
            pltpu.SemaphoreType.DMA((N_DEV,)),
            pltpu.SemaphoreType.DMA((N_DEV,)),
        ],
        compiler_params=pltpu.CompilerParams(collective_id=0),
    )(
        x.astype(jnp.bfloat16),
        Wq.astype(jnp.bfloat16),
        Wk.astype(jnp.bfloat16),
        Wv.astype(jnp.bfloat16),
        Wo.astype(jnp.bfloat16),
        jnp.asarray(cos_np),
        jnp.asarray(sin_np),
        jnp.asarray(p_np),
    )


# device time: 163289 ns/iter; 1.1901x vs baseline; 1.1127x over previous
import jax
import jax.numpy as jnp
import numpy as np
from jax import lax
from jax.experimental import pallas as pl
from jax.experimental.pallas import tpu as pltpu

N_DEV = 4
SQ = 1024
D = 1024
HQ = 8
DH = 128
SQH = SQ // 2
SCALE = 0.08838834764831843


def _rope_tables():
    inv = 1.0 / (10000.0 ** (np.arange(0, DH, 2) / DH))
    pos = np.arange(SQ)[:, None] * inv[None, :]
    cos = np.repeat(np.cos(pos), 2, axis=-1).astype(np.float32)
    sin = np.repeat(np.sin(pos), 2, axis=-1).astype(np.float32)
    P = np.zeros((DH, DH), np.float32)
    for k in range(DH // 2):
        P[2 * k + 1, 2 * k] = -1.0
        P[2 * k, 2 * k + 1] = 1.0
    return cos, sin, cos * SCALE, sin * SCALE, P


def _body(x_ref, wq_ref, wk_ref, wv_ref, wo_ref, cos_ref, sin_ref,
          cosq_ref, sinq_ref, p_ref,
          out_ref, xbuf, comm, ag_send, ag_recv, rs_send, rs_recv):
    i = lax.axis_index("i")
    right = lax.rem(i + 1, N_DEV)
    left = lax.rem(i + N_DEV - 1, N_DEV)

    barrier = pltpu.get_barrier_semaphore()
    for nbr in (left, right):
        pl.semaphore_signal(barrier, inc=1, device_id=(nbr,),
                            device_id_type=pl.DeviceIdType.MESH)
    pl.semaphore_wait(barrier, 2)

    ag = [
        pltpu.make_async_remote_copy(
            src_ref=xbuf.at[h],
            dst_ref=xbuf.at[h + 1],
            send_sem=ag_send.at[h],
            recv_sem=ag_recv.at[h],
            device_id=(right,),
            device_id_type=pl.DeviceIdType.MESH,
        )
        for h in range(N_DEV - 1)
    ]
    rs = [
        pltpu.make_async_remote_copy(
            src_ref=xbuf.at[h + 1],
            dst_ref=comm.at[h],
            send_sem=rs_send.at[h],
            recv_sem=rs_recv.at[h],
            device_id=(right,),
            device_id_type=pl.DeviceIdType.MESH,
        )
        for h in range(N_DEV - 2)
    ]
    rs_last = [
        pltpu.make_async_remote_copy(
            src_ref=xbuf.at[3, pl.ds(half * SQH, SQH)],
            dst_ref=comm.at[2, pl.ds(half * SQH, SQH)],
            send_sem=rs_send.at[2 + half],
            recv_sem=rs_recv.at[2 + half],
            device_id=(right,),
            device_id_type=pl.DeviceIdType.MESH,
        )
        for half in range(2)
    ]

    cos = cos_ref[...]
    sin = sin_ref[...]
    cosq = cosq_ref[...]
    sinq = sinq_ref[...]
    p = p_ref[...].astype(jnp.bfloat16)

    def rope_qkv(xb):
        q = jnp.dot(xb, wq_ref[...], preferred_element_type=jnp.float32)
        k = jnp.dot(xb, wk_ref[...], preferred_element_type=jnp.float32)
        v = jnp.dot(xb, wv_ref[...], preferred_element_type=jnp.float32)
        v = v.astype(jnp.bfloat16)
        qhs, khs = [], []
        for h in range(HQ):
            sl = slice(h * DH, (h + 1) * DH)
            qh32, kh32 = q[:, sl], k[:, sl]
            q_rot = jnp.dot(qh32.astype(jnp.bfloat16), p,
                            preferred_element_type=jnp.float32)
            k_rot = jnp.dot(kh32.astype(jnp.bfloat16), p,
                            preferred_element_type=jnp.float32)
            qhs.append((qh32 * cosq + q_rot * sinq).astype(jnp.bfloat16))
            khs.append((kh32 * cos + k_rot * sin).astype(jnp.bfloat16))
        return qhs, khs, v

    def attend(qhs, khs, v, rows, nrows):
        partial = jnp.zeros((nrows, D), jnp.float32)
        for h in range(HQ):
            sl = slice(h * DH, (h + 1) * DH)
            s_mat = lax.dot_general(
                qhs[h][rows], khs[h], (((1,), (1,)), ((), ())),
                preferred_element_type=jnp.float32)
            w = jnp.exp(s_mat)
            rsum = jnp.sum(w, axis=1, keepdims=True)
            ctx = jnp.dot(w.astype(jnp.bfloat16), v[:, sl],
                          preferred_element_type=jnp.float32)
            ctx = ctx * (1.0 / rsum)
            partial = partial + jnp.dot(ctx.astype(jnp.bfloat16),
                                        wo_ref[sl, :],
                                        preferred_element_type=jnp.float32)
        return partial

    def compute_slot(xb):
        qhs, khs, v = rope_qkv(xb)
        return attend(qhs, khs, v, slice(None), SQ)

    xbuf[0] = x_ref[0]
    ag[0].start()
    p0 = compute_slot(xbuf[0])
    ag[0].wait_send()
    xbuf[0] = p0.astype(jnp.bfloat16)

    ag[0].wait_recv()
    ag[1].start()
    p1 = compute_slot(xbuf[1])
    ag[1].wait_send()
    xbuf[1] = p1.astype(jnp.bfloat16)
    rs[0].start()

    ag[1].wait_recv()
    ag[2].start()
    p2 = compute_slot(xbuf[2])
    rs[0].wait_recv()
    acc2 = p2 + comm[0].astype(jnp.float32)
    ag[2].wait_send()
    xbuf[2] = acc2.astype(jnp.bfloat16)
    rs[1].start()

    ag[2].wait_recv()
    qhs3, khs3, v3 = rope_qkv(xbuf[3])
    p3a = attend(qhs3, khs3, v3, slice(0, SQH), SQH)
    rs[1].wait_recv()
    c1 = comm[1].astype(jnp.float32)
    xbuf[3, pl.ds(0, SQH)] = (p3a + c1[:SQH]).astype(jnp.bfloat16)
    rs_last[0].start()
    p3b = attend(qhs3, khs3, v3, slice(SQH, SQ), SQH)
    xbuf[3, pl.ds(SQH, SQH)] = (p3b + c1[SQH:]).astype(jnp.bfloat16)
    rs_last[1].start()

    rs_last[0].wait_recv()
    rs_last[1].wait_recv()
    out_ref[0, :, :] = (xbuf[0].astype(jnp.float32)
                        + comm[2].astype(jnp.float32))

    rs[0].wait_send()
    rs[1].wait_send()
    rs_last[0].wait_send()
    rs_last[1].wait_send()


def kernel(x, Wq, Wk, Wv, Wo):
    cos_np, sin_np, cosq_np, sinq_np, p_np = _rope_tables()
    return pl.pallas_call(
        _body,
        out_shape=jax.ShapeDtypeStruct((1, SQ, D), jnp.float32),
        in_specs=[pl.BlockSpec(memory_space=pltpu.VMEM)] * 10,
        out_specs=pl.BlockSpec(memory_space=pltpu.VMEM),
        scratch_shapes=[
            pltpu.VMEM((N_DEV, SQ, D), jnp.bfloat16),
            pltpu.VMEM((N_DEV - 1, SQ, D), jnp.bfloat16),
            pltpu.SemaphoreType.DMA((N_DEV - 1,)),
            pltpu.SemaphoreType.DMA((N_DEV - 1,)),
            pltpu.SemaphoreType.DMA((N_DEV,)),
            pltpu.SemaphoreType.DMA((N_DEV,)),
        ],
        compiler_params=pltpu.CompilerParams(collective_id=0),
    )(
        x.astype(jnp.bfloat16),
        Wq.astype(jnp.bfloat16),
        Wk.astype(jnp.bfloat16),
        Wv.astype(jnp.bfloat16),
        Wo.astype(jnp.bfloat16),
        jnp.asarray(cos_np),
        jnp.asarray(sin_np),
        jnp.asarray(cosq_np),
        jnp.asarray(sinq_np),
        jnp.asarray(p_np),
    )


# device time: 162231 ns/iter; 1.1979x vs baseline; 1.0065x over previous
import jax
import jax.numpy as jnp
import numpy as np
from jax import lax
from jax.experimental import pallas as pl
from jax.experimental.pallas import tpu as pltpu

N_DEV = 4
SQ = 1024
D = 1024
HQ = 8
DH = 128
SQH = SQ // 2
SQQ = SQ // 4
SCALE = 0.08838834764831843


def _rope_tables():
    inv = 1.0 / (10000.0 ** (np.arange(0, DH, 2) / DH))
    pos = np.arange(SQ)[:, None] * inv[None, :]
    cos = np.repeat(np.cos(pos), 2, axis=-1).astype(np.float32)
    sin = np.repeat(np.sin(pos), 2, axis=-1).astype(np.float32)
    P = np.zeros((DH, DH), np.float32)
    for k in range(DH // 2):
        P[2 * k + 1, 2 * k] = -1.0
        P[2 * k, 2 * k + 1] = 1.0
    return cos, sin, cos * SCALE, sin * SCALE, P


def _body(x_ref, wq_ref, wk_ref, wv_ref, wo_ref, cos_ref, sin_ref,
          cosq_ref, sinq_ref, p_ref,
          out_ref, xbuf, comm, ag_send, ag_recv, rs_send, rs_recv):
    i = lax.axis_index("i")
    right = lax.rem(i + 1, N_DEV)
    left = lax.rem(i + N_DEV - 1, N_DEV)

    barrier = pltpu.get_barrier_semaphore()
    for nbr in (left, right):
        pl.semaphore_signal(barrier, inc=1, device_id=(nbr,),
                            device_id_type=pl.DeviceIdType.MESH)
    pl.semaphore_wait(barrier, 2)

    ag = [
        pltpu.make_async_remote_copy(
            src_ref=xbuf.at[h],
            dst_ref=xbuf.at[h + 1],
            send_sem=ag_send.at[h],
            recv_sem=ag_recv.at[h],
            device_id=(right,),
            device_id_type=pl.DeviceIdType.MESH,
        )
        for h in range(N_DEV - 1)
    ]
    rs = [
        pltpu.make_async_remote_copy(
            src_ref=xbuf.at[h + 1],
            dst_ref=comm.at[h],
            send_sem=rs_send.at[h],
            recv_sem=rs_recv.at[h],
            device_id=(right,),
            device_id_type=pl.DeviceIdType.MESH,
        )
        for h in range(N_DEV - 2)
    ]
    rs_last = [
        pltpu.make_async_remote_copy(
            src_ref=xbuf.at[3, pl.ds(quarter * SQQ, SQQ)],
            dst_ref=comm.at[2, pl.ds(quarter * SQQ, SQQ)],
            send_sem=rs_send.at[2 + quarter],
            recv_sem=rs_recv.at[2 + quarter],
            device_id=(right,),
            device_id_type=pl.DeviceIdType.MESH,
        )
        for quarter in range(4)
    ]

    cos = cos_ref[...]
    sin = sin_ref[...]
    cosq = cosq_ref[...]
    sinq = sinq_ref[...]
    p = p_ref[...].astype(jnp.bfloat16)

    def rope_qkv(xb):
        q = jnp.dot(xb, wq_ref[...], preferred_element_type=jnp.float32)
        k = jnp.dot(xb, wk_ref[...], preferred_element_type=jnp.float32)
        v = jnp.dot(xb, wv_ref[...], preferred_element_type=jnp.float32)
        v = v.astype(jnp.bfloat16)
        qhs, khs = [], []
        for h in range(HQ):
            sl = slice(h * DH, (h + 1) * DH)
            qh32, kh32 = q[:, sl], k[:, sl]
            q_rot = jnp.dot(qh32.astype(jnp.bfloat16), p,
                            preferred_element_type=jnp.float32)
            k_rot = jnp.dot(kh32.astype(jnp.bfloat16), p,
                            preferred_element_type=jnp.float32)
            qhs.append((qh32 * cosq + q_rot * sinq).astype(jnp.bfloat16))
            khs.append((kh32 * cos + k_rot * sin).astype(jnp.bfloat16))
        return qhs, khs, v

    def attend(qhs, khs, v, rows, nrows):
        partial = jnp.zeros((nrows, D), jnp.float32)
        for h in range(HQ):
            sl = slice(h * DH, (h + 1) * DH)
            s_mat = lax.dot_general(
                qhs[h][rows], khs[h], (((1,), (1,)), ((), ())),
                preferred_element_type=jnp.float32)
            w = jnp.exp(s_mat)
            rsum = jnp.sum(w, axis=1, keepdims=True)
            ctx = jnp.dot(w.astype(jnp.bfloat16), v[:, sl],
                          preferred_element_type=jnp.float32)
            ctx = ctx * (1.0 / rsum)
            partial = partial + jnp.dot(ctx.astype(jnp.bfloat16),
                                        wo_ref[sl, :],
                                        preferred_element_type=jnp.float32)
        return partial

    def compute_slot(xb):
        qhs, khs, v = rope_qkv(xb)
        return attend(qhs, khs, v, slice(None), SQ)

    xbuf[0] = x_ref[0]
    ag[0].start()
    p0 = compute_slot(xbuf[0])
    ag[0].wait_send()
    xbuf[0] = p0.astype(jnp.bfloat16)

    ag[0].wait_recv()
    ag[1].start()
    p1 = compute_slot(xbuf[1])
    ag[1].wait_send()
    xbuf[1] = p1.astype(jnp.bfloat16)
    rs[0].start()

    ag[1].wait_recv()
    ag[2].start()
    p2 = compute_slot(xbuf[2])
    rs[0].wait_recv()
    acc2 = p2 + comm[0].astype(jnp.float32)
    ag[2].wait_send()
    xbuf[2] = acc2.astype(jnp.bfloat16)
    rs[1].start()

    ag[2].wait_recv()
    qhs3, khs3, v3 = rope_qkv(xbuf[3])
    for quarter in range(4):
        rows = slice(quarter * SQQ, (quarter + 1) * SQQ)
        p3q = attend(qhs3, khs3, v3, rows, SQQ)
        if quarter == 0:
            rs[1].wait_recv()
        c1q = comm[1, rows].astype(jnp.float32)
        xbuf[3, pl.ds(quarter * SQQ, SQQ)] = (p3q + c1q).astype(jnp.bfloat16)
        rs_last[quarter].start()

    for quarter in range(4):
        rows = slice(quarter * SQQ, (quarter + 1) * SQQ)
        rs_last[quarter].wait_recv()
        out_ref[0, rows, :] = (xbuf[0, rows].astype(jnp.float32)
                               + comm[2, rows].astype(jnp.float32))

    rs[0].wait_send()
    rs[1].wait_send()
    for quarter in range(4):
        rs_last[quarter].wait_send()


def kernel(x, Wq, Wk, Wv, Wo):
    cos_np, sin_np, cosq_np, sinq_np, p_np = _rope_tables()
    return pl.pallas_call(
        _body,
        out_shape=jax.ShapeDtypeStruct((1, SQ, D), jnp.float32),
        in_specs=[pl.BlockSpec(memory_space=pltpu.VMEM)] * 10,
        out_specs=pl.BlockSpec(memory_space=pltpu.VMEM),
        scratch_shapes=[
            pltpu.VMEM((N_DEV, SQ, D), jnp.bfloat16),
            pltpu.VMEM((N_DEV - 1, SQ, D), jnp.bfloat16),
            pltpu.SemaphoreType.DMA((N_DEV - 1,)),
            pltpu.SemaphoreType.DMA((N_DEV - 1,)),
            pltpu.SemaphoreType.DMA((6,)),
            pltpu.SemaphoreType.DMA((6,)),
        ],
        compiler_params=pltpu.CompilerParams(collective_id=0),
    )(
        x.astype(jnp.bfloat16),
        Wq.astype(jnp.bfloat16),
        Wk.astype(jnp.bfloat16),
        Wv.astype(jnp.bfloat16),
        Wo.astype(jnp.bfloat16),
        jnp.asarray(cos_np),
        jnp.asarray(sin_np),
        jnp.asarray(cosq_np),
        jnp.asarray(sinq_np),
        jnp.asarray(p_np),
    )


# device time: 139335 ns/iter; 1.3947x vs baseline; 1.1643x over previous
import jax
import jax.numpy as jnp
import numpy as np
from jax import lax
from jax.experimental import pallas as pl
from jax.experimental.pallas import tpu as pltpu

N_DEV = 4
SQ = 1024
D = 1024
HQ = 8
DH = 128
SQH = SQ // 2
SQQ = SQ // 4
SCALE = 0.08838834764831843


def _rope_tables():
    inv = 1.0 / (10000.0 ** (np.arange(0, DH, 2) / DH))
    pos = np.arange(SQ)[:, None] * inv[None, :]
    cos = np.repeat(np.cos(pos), 2, axis=-1).astype(np.float32)
    sin = np.repeat(np.sin(pos), 2, axis=-1).astype(np.float32)
    P = np.zeros((DH, DH), np.float32)
    for k in range(DH // 2):
        P[2 * k + 1, 2 * k] = -1.0
        P[2 * k, 2 * k + 1] = 1.0
    return cos, sin, cos * SCALE, sin * SCALE, P


def _body(x_ref, wq_ref, wk_ref, wv_ref, wo_ref, cos_ref, sin_ref,
          cosq_ref, sinq_ref, p_ref,
          out_ref, xbuf, comm, ag_send, ag_recv, rs_send, rs_recv):
    i = lax.axis_index("i")
    right = lax.rem(i + 1, N_DEV)
    left = lax.rem(i + N_DEV - 1, N_DEV)
    diag = lax.rem(i + 2, N_DEV)

    barrier = pltpu.get_barrier_semaphore()
    for nbr in (left, right, diag):
        pl.semaphore_signal(barrier, inc=1, device_id=(nbr,),
                            device_id_type=pl.DeviceIdType.MESH)
    pl.semaphore_wait(barrier, 3)

    ag = [
        pltpu.make_async_remote_copy(
            src_ref=xbuf.at[h],
            dst_ref=xbuf.at[h + 1],
            send_sem=ag_send.at[h],
            recv_sem=ag_recv.at[h],
            device_id=(right,),
            device_id_type=pl.DeviceIdType.MESH,
        )
        for h in range(N_DEV - 1)
    ]
    rs_l = pltpu.make_async_remote_copy(
        src_ref=xbuf.at[1],
        dst_ref=comm.at[0],
        send_sem=rs_send.at[0],
        recv_sem=rs_recv.at[0],
        device_id=(left,),
        device_id_type=pl.DeviceIdType.MESH,
    )
    rs_d = pltpu.make_async_remote_copy(
        src_ref=xbuf.at[2],
        dst_ref=comm.at[1],
        send_sem=rs_send.at[1],
        recv_sem=rs_recv.at[1],
        device_id=(diag,),
        device_id_type=pl.DeviceIdType.MESH,
    )
    rs_r = [
        pltpu.make_async_remote_copy(
            src_ref=xbuf.at[3, pl.ds(quarter * SQQ, SQQ)],
            dst_ref=comm.at[2, pl.ds(quarter * SQQ, SQQ)],
            send_sem=rs_send.at[2 + quarter],
            recv_sem=rs_recv.at[2 + quarter],
            device_id=(right,),
            device_id_type=pl.DeviceIdType.MESH,
        )
        for quarter in range(4)
    ]

    cos = cos_ref[...]
    sin = sin_ref[...]
    cosq = cosq_ref[...]
    sinq = sinq_ref[...]
    p = p_ref[...].astype(jnp.bfloat16)

    def rope_qkv(xb):
        q = jnp.dot(xb, wq_ref[...], preferred_element_type=jnp.float32)
        k = jnp.dot(xb, wk_ref[...], preferred_element_type=jnp.float32)
        v = jnp.dot(xb, wv_ref[...], preferred_element_type=jnp.float32)
        v = v.astype(jnp.bfloat16)
        qhs, khs = [], []
        for h in range(HQ):
            sl = slice(h * DH, (h + 1) * DH)
            qh32, kh32 = q[:, sl], k[:, sl]
            q_rot = jnp.dot(qh32.astype(jnp.bfloat16), p,
                            preferred_element_type=jnp.float32)
            k_rot = jnp.dot(kh32.astype(jnp.bfloat16), p,
                            preferred_element_type=jnp.float32)
            qhs.append((qh32 * cosq + q_rot * sinq).astype(jnp.bfloat16))
            khs.append((kh32 * cos + k_rot * sin).astype(jnp.bfloat16))
        return qhs, khs, v

    def attend(qhs, khs, v, rows, nrows):
        partial = jnp.zeros((nrows, D), jnp.float32)
        for h in range(HQ):
            sl = slice(h * DH, (h + 1) * DH)
            s_mat = lax.dot_general(
                qhs[h][rows], khs[h], (((1,), (1,)), ((), ())),
                preferred_element_type=jnp.float32)
            w = jnp.exp(s_mat)
            rsum = jnp.sum(w, axis=1, keepdims=True)
            ctx = jnp.dot(w.astype(jnp.bfloat16), v[:, sl],
                          preferred_element_type=jnp.float32)
            ctx = ctx * (1.0 / rsum)
            partial = partial + jnp.dot(ctx.astype(jnp.bfloat16),
                                        wo_ref[sl, :],
                                        preferred_element_type=jnp.float32)
        return partial

    def compute_slot(xb):
        qhs, khs, v = rope_qkv(xb)
        return attend(qhs, khs, v, slice(None), SQ)

    xbuf[0] = x_ref[0]
    ag[0].start()
    p0 = compute_slot(xbuf[0])
    ag[0].wait_send()
    xbuf[0] = p0.astype(jnp.bfloat16)

    ag[0].wait_recv()
    ag[1].start()
    p1 = compute_slot(xbuf[1])
    ag[1].wait_send()
    xbuf[1] = p1.astype(jnp.bfloat16)
    rs_l.start()

    ag[1].wait_recv()
    ag[2].start()
    p2 = compute_slot(xbuf[2])
    ag[2].wait_send()
    xbuf[2] = p2.astype(jnp.bfloat16)
    rs_d.start()

    ag[2].wait_recv()
    qhs3, khs3, v3 = rope_qkv(xbuf[3])
    for quarter in range(4):
        rows = slice(quarter * SQQ, (quarter + 1) * SQQ)
        p3q = attend(qhs3, khs3, v3, rows, SQQ)
        xbuf[3, pl.ds(quarter * SQQ, SQQ)] = p3q.astype(jnp.bfloat16)
        rs_r[quarter].start()

    rs_l.wait_recv()
    rs_d.wait_recv()
    for quarter in range(4):
        rows = slice(quarter * SQQ, (quarter + 1) * SQQ)
        rs_r[quarter].wait_recv()
        out_ref[0, rows, :] = (xbuf[0, rows].astype(jnp.float32)
                               + comm[0, rows].astype(jnp.float32)
                               + comm[1, rows].astype(jnp.float32)
                               + comm[2, rows].astype(jnp.float32))

    rs_l.wait_send()
    rs_d.wait_send()
    for quarter in range(4):
        rs_r[quarter].wait_send()


def kernel(x, Wq, Wk, Wv, Wo):
    cos_np, sin_np, cosq_np, sinq_np, p_np = _rope_tables()
    return pl.pallas_call(
        _body,
        out_shape=jax.ShapeDtypeStruct((1, SQ, D), jnp.float32),
        in_specs=[pl.BlockSpec(memory_space=pltpu.VMEM)] * 10,
        out_specs=pl.BlockSpec(memory_space=pltpu.VMEM),
        scratch_shapes=[
            pltpu.VMEM((N_DEV, SQ, D), jnp.bfloat16),
            pltpu.VMEM((N_DEV - 1, SQ, D), jnp.bfloat16),
            pltpu.SemaphoreType.DMA((N_DEV - 1,)),
            pltpu.SemaphoreType.DMA((N_DEV - 1,)),
            pltpu.SemaphoreType.DMA((6,)),
            pltpu.SemaphoreType.DMA((6,)),
        ],
        compiler_params=pltpu.CompilerParams(collective_id=0),
    )(
        x.astype(jnp.bfloat16),
        Wq.astype(jnp.bfloat16),
        Wk.astype(jnp.bfloat16),
        Wv.astype(jnp.bfloat16),
        Wo.astype(jnp.bfloat16),
        jnp.asarray(cos_np),
        jnp.asarray(sin_np),
        jnp.asarray(cosq_np),
        jnp.asarray(sinq_np),
        jnp.asarray(p_np),
    )


# device time: 139313 ns/iter; 1.3950x vs baseline; 1.0002x over previous
import jax
import jax.numpy as jnp
import numpy as np
from jax import lax
from jax.experimental import pallas as pl
from jax.experimental.pallas import tpu as pltpu

N_DEV = 4
SQ = 1024
D = 1024
HQ = 8
DH = 128
SQH = SQ // 2
SQQ = SQ // 4
SQ8 = SQ // 8
SCALE = 0.08838834764831843


def _rope_tables():
    inv = 1.0 / (10000.0 ** (np.arange(0, DH, 2) / DH))
    pos = np.arange(SQ)[:, None] * inv[None, :]
    cos = np.repeat(np.cos(pos), 2, axis=-1).astype(np.float32)
    sin = np.repeat(np.sin(pos), 2, axis=-1).astype(np.float32)
    P = np.zeros((DH, DH), np.float32)
    for k in range(DH // 2):
        P[2 * k + 1, 2 * k] = -1.0
        P[2 * k, 2 * k + 1] = 1.0
    return cos, sin, cos * SCALE, sin * SCALE, P


def _body(x_ref, wq_ref, wk_ref, wv_ref, wo_ref, cos_ref, sin_ref,
          cosq_ref, sinq_ref, p_ref,
          out_ref, xbuf, comm, ag_send, ag_recv, rs_send, rs_recv):
    i = lax.axis_index("i")
    right = lax.rem(i + 1, N_DEV)
    left = lax.rem(i + N_DEV - 1, N_DEV)
    diag = lax.rem(i + 2, N_DEV)

    barrier = pltpu.get_barrier_semaphore()
    for nbr in (left, right, diag):
        pl.semaphore_signal(barrier, inc=1, device_id=(nbr,),
                            device_id_type=pl.DeviceIdType.MESH)
    pl.semaphore_wait(barrier, 3)

    ag = [
        pltpu.make_async_remote_copy(
            src_ref=xbuf.at[h],
            dst_ref=xbuf.at[h + 1],
            send_sem=ag_send.at[h],
            recv_sem=ag_recv.at[h],
            device_id=(right,),
            device_id_type=pl.DeviceIdType.MESH,
        )
        for h in range(N_DEV - 1)
    ]
    rs_l = pltpu.make_async_remote_copy(
        src_ref=xbuf.at[1],
        dst_ref=comm.at[0],
        send_sem=rs_send.at[0],
        recv_sem=rs_recv.at[0],
        device_id=(left,),
        device_id_type=pl.DeviceIdType.MESH,
    )
    rs_d = pltpu.make_async_remote_copy(
        src_ref=xbuf.at[2],
        dst_ref=comm.at[1],
        send_sem=rs_send.at[1],
        recv_sem=rs_recv.at[1],
        device_id=(diag,),
        device_id_type=pl.DeviceIdType.MESH,
    )
    rs_r = [
        pltpu.make_async_remote_copy(
            src_ref=xbuf.at[3, pl.ds(piece * SQ8, SQ8)],
            dst_ref=comm.at[2, pl.ds(piece * SQ8, SQ8)],
            send_sem=rs_send.at[2 + piece],
            recv_sem=rs_recv.at[2 + piece],
            device_id=(right,),
            device_id_type=pl.DeviceIdType.MESH,
        )
        for piece in range(8)
    ]

    cos = cos_ref[...]
    sin = sin_ref[...]
    cosq = cosq_ref[...]
    sinq = sinq_ref[...]
    p = p_ref[...].astype(jnp.bfloat16)

    def rope_qkv(xb):
        q = jnp.dot(xb, wq_ref[...], preferred_element_type=jnp.float32)
        k = jnp.dot(xb, wk_ref[...], preferred_element_type=jnp.float32)
        v = jnp.dot(xb, wv_ref[...], preferred_element_type=jnp.float32)
        v = v.astype(jnp.bfloat16)
        qhs, khs = [], []
        for h in range(HQ):
            sl = slice(h * DH, (h + 1) * DH)
            qh32, kh32 = q[:, sl], k[:, sl]
            q_rot = jnp.dot(qh32.astype(jnp.bfloat16), p,
                            preferred_element_type=jnp.float32)
            k_rot = jnp.dot(kh32.astype(jnp.bfloat16), p,
                            preferred_element_type=jnp.float32)
            qhs.append((qh32 * cosq + q_rot * sinq).astype(jnp.bfloat16))
            khs.append((kh32 * cos + k_rot * sin).astype(jnp.bfloat16))
        return qhs, khs, v

    def attend(qhs, khs, v, rows, nrows):
        partial = jnp.zeros((nrows, D), jnp.float32)
        for h in range(HQ):
            sl = slice(h * DH, (h + 1) * DH)
            s_mat = lax.dot_general(
                qhs[h][rows], khs[h], (((1,), (1,)), ((), ())),
                preferred_element_type=jnp.float32)
            w = jnp.exp(s_mat)
            rsum = jnp.sum(w, axis=1, keepdims=True)
            ctx = jnp.dot(w.astype(jnp.bfloat16), v[:, sl],
                          preferred_element_type=jnp.float32)
            ctx = ctx * (1.0 / rsum)
            partial = partial + jnp.dot(ctx.astype(jnp.bfloat16),
                                        wo_ref[sl, :],
                                        preferred_element_type=jnp.float32)
        return partial

    def compute_slot(xb):
        qhs, khs, v = rope_qkv(xb)
        return attend(qhs, khs, v, slice(None), SQ)

    xbuf[0] = x_ref[0]
    ag[0].start()
    p0 = compute_slot(xbuf[0])
    ag[0].wait_send()
    xbuf[0] = p0.astype(jnp.bfloat16)

    ag[0].wait_recv()
    ag[1].start()
    p1 = compute_slot(xbuf[1])
    ag[1].wait_send()
    xbuf[1] = p1.astype(jnp.bfloat16)
    rs_l.start()

    ag[1].wait_recv()
    ag[2].start()
    p2 = compute_slot(xbuf[2])
    ag[2].wait_send()
    xbuf[2] = p2.astype(jnp.bfloat16)
    rs_d.start()

    ag[2].wait_recv()
    qhs3, khs3, v3 = rope_qkv(xbuf[3])
    for piece in range(8):
        rows = slice(piece * SQ8, (piece + 1) * SQ8)
        p3q = attend(qhs3, khs3, v3, rows, SQ8)
        xbuf[3, pl.ds(piece * SQ8, SQ8)] = p3q.astype(jnp.bfloat16)
        rs_r[piece].start()

    rs_l.wait_recv()
    rs_d.wait_recv()
    for piece in range(8):
        rows = slice(piece * SQ8, (piece + 1) * SQ8)
        rs_r[piece].wait_recv()
        out_ref[0, rows, :] = (xbuf[0, rows].astype(jnp.float32)
                               + comm[0, rows].astype(jnp.float32)
                               + comm[1, rows].astype(jnp.float32)
                               + comm[2, rows].astype(jnp.float32))

    rs_l.wait_send()
    rs_d.wait_send()
    for piece in range(8):
        rs_r[piece].wait_send()


def kernel(x, Wq, Wk, Wv, Wo):
    cos_np, sin_np, cosq_np, sinq_np, p_np = _rope_tables()
    return pl.pallas_call(
        _body,
        out_shape=jax.ShapeDtypeStruct((1, SQ, D), jnp.float32),
        in_specs=[pl.BlockSpec(memory_space=pltpu.VMEM)] * 10,
        out_specs=pl.BlockSpec(memory_space=pltpu.VMEM),
        scratch_shapes=[
            pltpu.VMEM((N_DEV, SQ, D), jnp.bfloat16),
            pltpu.VMEM((N_DEV - 1, SQ, D), jnp.bfloat16),
            pltpu.SemaphoreType.DMA((N_DEV - 1,)),
            pltpu.SemaphoreType.DMA((N_DEV - 1,)),
            pltpu.SemaphoreType.DMA((10,)),
            pltpu.SemaphoreType.DMA((10,)),
        ],
        compiler_params=pltpu.CompilerParams(collective_id=0),
    )(
        x.astype(jnp.bfloat16),
        Wq.astype(jnp.bfloat16),
        Wk.astype(jnp.bfloat16),
        Wv.astype(jnp.bfloat16),
        Wo.astype(jnp.bfloat16),
        jnp.asarray(cos_np),
        jnp.asarray(sin_np),
        jnp.asarray(cosq_np),
        jnp.asarray(sinq_np),
        jnp.asarray(p_np),
    )
